# Initial kernel scaffold; baseline (speedup 1.0000x reference)
import jax
import jax.numpy as jnp
from jax.experimental import pallas as pl


def kernel(x, edge_index, W1, b1, W2, b2, W3, b3):
    raise NotImplementedError("write your pallas kernel here")


# speedup vs baseline: 9.8852x; 9.8852x over previous
import functools

import jax
import jax.numpy as jnp
from jax import lax
from jax.experimental import pallas as pl
from jax.experimental.pallas import tpu as pltpu
from jax.experimental.pallas import tpu_sc as plsc

N = 10000
E = 320000
NC = 2
NS = 16
B = 128
NP = 10240
SPT = NP // NS
RB = 1000
NBUF = 1
NB1 = -(-(E // (NC * NS)) // B)
EPAD = NC * NS * NB1 * B - E

_mesh = plsc.VectorSubcoreMesh(
    core_axis_name="c", subcore_axis_name="s", num_cores=NC, num_subcores=NS)


def _deg_body(dst_hbm, zeros_hbm, ones_hbm, hist_out, dst_v, ones_v, acc, sem):
    c = lax.axis_index("c")
    s = lax.axis_index("s")
    row0 = s * SPT
    pltpu.sync_copy(zeros_hbm.at[pl.ds(row0, SPT)], acc.at[pl.ds(row0, SPT)])
    pltpu.sync_copy(dst_hbm.at[c, s], dst_v)
    pltpu.sync_copy(ones_hbm, ones_v)
    plsc.subcore_barrier()

    def body(j, carry):
        d = pltpu.make_async_copy(ones_v, acc.at[dst_v.at[j]], sem)
        d.start(add=True)
        d.wait()
        return carry

    lax.fori_loop(0, NB1, body, 0)
    plsc.subcore_barrier()
    pltpu.sync_copy(acc.at[pl.ds(row0, SPT)], hist_out.at[c, pl.ds(row0, SPT)])


_deg_call = pl.kernel(
    _deg_body,
    out_type=jax.ShapeDtypeStruct((NC, NP, 128), jnp.float32),
    mesh=_mesh,
    scratch_types=[
        pltpu.VMEM((NB1, B), jnp.int32),
        pltpu.VMEM((B, 128), jnp.float32),
        pltpu.VMEM_SHARED((NP, 128), jnp.float32),
        pltpu.SemaphoreType.DMA,
    ],
)


def _agg_body(nb, tables, src_hbm, dst_hbm, zeros_hbm, out,
              src_v, dst_v, bufs, acc, gsem, ssem):
    c = lax.axis_index("c")
    s = lax.axis_index("s")
    row0 = s * SPT
    pltpu.sync_copy(zeros_hbm.at[pl.ds(row0, SPT)], acc.at[pl.ds(row0, SPT)])
    pltpu.sync_copy(src_hbm.at[c, s], src_v)
    pltpu.sync_copy(dst_hbm.at[c, s], dst_v)
    plsc.subcore_barrier()
    table = tables

    def body(j, carry):
        d = pltpu.make_async_copy(table.at[src_v.at[j]], bufs.at[0], gsem.at[0])
        d.start()
        d.wait()
        d2 = pltpu.make_async_copy(bufs.at[0], acc.at[dst_v.at[j]], ssem.at[0])
        d2.start(add=True)
        d2.wait()
        return carry

    lax.fori_loop(0, nb, body, 0)
    plsc.subcore_barrier()
    pltpu.sync_copy(acc.at[pl.ds(row0, SPT)], out.at[c, pl.ds(row0, SPT)])


def _make_agg(nb):
    return pl.kernel(
        functools.partial(_agg_body, nb),
        out_type=jax.ShapeDtypeStruct((NC, NP, 128), jnp.float32),
        mesh=_mesh,
        scratch_types=[
            pltpu.VMEM((nb, B), jnp.int32),
            pltpu.VMEM((nb, B), jnp.int32),
            pltpu.VMEM((NBUF, B, 128), jnp.float32),
            pltpu.VMEM_SHARED((NP, 128), jnp.float32),
            pltpu.SemaphoreType.DMA((NBUF,)),
            pltpu.SemaphoreType.DMA((NBUF,)),
        ],
    )


_agg_call = _make_agg(NB1)


def _prep_body(h0_ref, h1_ref, x_ref, dis_ref, inv_ref, xs_ref):
    deg = h0_ref[:, 0:1] + h1_ref[:, 0:1] + 1.0
    dis = lax.rsqrt(deg)
    dis_ref[...] = dis
    inv_ref[...] = 1.0 / deg
    xs_ref[...] = x_ref[...] * dis


def _prep_call(hist0, hist1, x):
    return pl.pallas_call(
        _prep_body,
        grid=(N // RB,),
        in_specs=[
            pl.BlockSpec((RB, 128), lambda r: (r, 0)),
            pl.BlockSpec((RB, 128), lambda r: (r, 0)),
            pl.BlockSpec((RB, 128), lambda r: (r, 0)),
        ],
        out_specs=[
            pl.BlockSpec((RB, 1), lambda r: (r, 0)),
            pl.BlockSpec((RB, 1), lambda r: (r, 0)),
            pl.BlockSpec((RB, 128), lambda r: (r, 0)),
        ],
        out_shape=[
            jax.ShapeDtypeStruct((N, 1), jnp.float32),
            jax.ShapeDtypeStruct((N, 1), jnp.float32),
            jax.ShapeDtypeStruct((N, 128), jnp.float32),
        ],
    )(hist0, hist1, x)


def _layer1_body(part_ref, x_ref, dis_ref, inv_ref, w1_ref, b1_ref,
                 g_ref, g2_ref, ih_ref):
    dis = dis_ref[...]
    inv = inv_ref[...]
    a = dis * (part_ref[0] + part_ref[1]) + inv * x_ref[...]
    h1 = jnp.maximum(
        jnp.dot(a, w1_ref[...], preferred_element_type=jnp.float32)
        + b1_ref[...], 0.0)
    g = h1 * dis
    g_ref[...] = g[:, :128]
    g2_ref[...] = g[:, 128:]
    ih_ref[...] = h1 * inv


def _layer1_call(part1, x, dis, inv, W1, b1r):
    return pl.pallas_call(
        _layer1_body,
        grid=(N // RB,),
        in_specs=[
            pl.BlockSpec((2, RB, 128), lambda r: (0, r, 0)),
            pl.BlockSpec((RB, 128), lambda r: (r, 0)),
            pl.BlockSpec((RB, 1), lambda r: (r, 0)),
            pl.BlockSpec((RB, 1), lambda r: (r, 0)),
            pl.BlockSpec((128, 256), lambda r: (0, 0)),
            pl.BlockSpec((1, 256), lambda r: (0, 0)),
        ],
        out_specs=[
            pl.BlockSpec((RB, 128), lambda r: (r, 0)),
            pl.BlockSpec((RB, 128), lambda r: (r, 0)),
            pl.BlockSpec((RB, 256), lambda r: (r, 0)),
        ],
        out_shape=[
            jax.ShapeDtypeStruct((N, 128), jnp.float32),
            jax.ShapeDtypeStruct((N, 128), jnp.float32),
            jax.ShapeDtypeStruct((N, 256), jnp.float32),
        ],
    )(part1, x, dis, inv, W1, b1r)


def _layer2_body(parta_ref, partb_ref, ih_ref, dis_ref, w2_ref, b2_ref,
                 w3_ref, b3_ref, out_ref):
    a2 = (dis_ref[...]
          * jnp.concatenate([parta_ref[0] + parta_ref[1],
                             partb_ref[0] + partb_ref[1]], axis=1)
          + ih_ref[...])
    h2 = jnp.maximum(
        jnp.dot(a2, w2_ref[...], preferred_element_type=jnp.float32)
        + b2_ref[...], 0.0)
    out_ref[...] = jax.nn.sigmoid(
        jnp.dot(h2, w3_ref[...], preferred_element_type=jnp.float32)
        + b3_ref[...])


def _layer2_call(part2a, part2b, ih1, dis, W2, b2r, W3, b3r):
    return pl.pallas_call(
        _layer2_body,
        grid=(N // RB,),
        in_specs=[
            pl.BlockSpec((2, RB, 128), lambda r: (0, r, 0)),
            pl.BlockSpec((2, RB, 128), lambda r: (0, r, 0)),
            pl.BlockSpec((RB, 256), lambda r: (r, 0)),
            pl.BlockSpec((RB, 1), lambda r: (r, 0)),
            pl.BlockSpec((256, 256), lambda r: (0, 0)),
            pl.BlockSpec((1, 256), lambda r: (0, 0)),
            pl.BlockSpec((256, 128), lambda r: (0, 0)),
            pl.BlockSpec((1, 128), lambda r: (0, 0)),
        ],
        out_specs=pl.BlockSpec((RB, 128), lambda r: (r, 0)),
        out_shape=jax.ShapeDtypeStruct((N, 128), jnp.float32),
    )(part2a, part2b, ih1, dis, W2, b2r, W3, b3r)


def kernel(x, edge_index, W1, b1, W2, b2, W3, b3):
    src = edge_index[0].astype(jnp.int32)
    dst = edge_index[1].astype(jnp.int32)
    src = jnp.concatenate([src, jnp.zeros((EPAD,), jnp.int32)])
    dst = jnp.concatenate([dst, jnp.full((EPAD,), N, jnp.int32)])
    src1 = src.reshape(NC, NS, NB1, B)
    dst1 = dst.reshape(NC, NS, NB1, B)
    zeros128 = jnp.zeros((NP, 128), jnp.float32)
    ones128 = jnp.ones((B, 128), jnp.float32)

    hist = _deg_call(dst1, zeros128, ones128)
    dis, inv, xs = _prep_call(hist[0], hist[1], x)
    part1 = _agg_call(xs, src1, dst1, zeros128)
    g0, g1, ih1 = _layer1_call(part1, x, dis, inv, W1, b1.reshape(1, 256))
    part2a = _agg_call(g0, src1, dst1, zeros128)
    part2b = _agg_call(g1, src1, dst1, zeros128)
    return _layer2_call(part2a, part2b, ih1, dis, W2, b2.reshape(1, 256),
                        W3, b3.reshape(1, 128))
